# Initial kernel scaffold; baseline (speedup 1.0000x reference)
import jax
import jax.numpy as jnp
from jax.experimental import pallas as pl


def kernel(prediction, labels):
    raise NotImplementedError("write your pallas kernel here")


# speedup vs baseline: 61.7015x; 61.7015x over previous
import functools

import jax
import jax.numpy as jnp
from jax import lax
from jax.experimental import pallas as pl
from jax.experimental.pallas import tpu as pltpu
from jax.experimental.pallas import tpu_sc as plsc

_C = 20
_B = 1024
_T = 2 * _C * _B
_LANES = 128
_ROWS = 240 * 180 * 32 // _LANES
_P = _ROWS * _LANES
_NB = 120
_GRID1 = _ROWS // _NB
_W = 32
_PER_W = _C * _P // _W
_K = 8640
_NCH = _PER_W // _K


def _tc_prep_body(x_ref, lab_ref, idx_ref, nll_ref):
    x = x_ref[...]
    lab = lab_ref[...]
    m = jnp.max(x, axis=0)
    ex = jnp.exp(x - m[None])
    s = jnp.sum(ex, axis=0)
    rinv = 1.0 / s
    lse = m + jnp.log(s)
    xl = jnp.zeros_like(m)
    for c in range(_C):
        xl = jnp.where(lab == c, x[c], xl)

    @pl.when(pl.program_id(0) == 0)
    def _():
        nll_ref[...] = jnp.zeros((1, 1), jnp.float32)

    nll_ref[...] += jnp.sum(lse - xl).reshape(1, 1)

    for c in range(_C):
        p = ex[c] * rinv
        fg = lab == c
        e = jnp.where(fg, 1.0 - p, p)
        b = jnp.minimum((e * _B).astype(jnp.int32), _B - 1)
        idx_ref[c] = c * _B + b + jnp.where(fg, _C * _B, 0)


def _tc_prep(logits, lab):
    return pl.pallas_call(
        _tc_prep_body,
        grid=(_GRID1,),
        in_specs=[
            pl.BlockSpec((_C, _NB, _LANES), lambda i: (0, i, 0)),
            pl.BlockSpec((_NB, _LANES), lambda i: (i, 0)),
        ],
        out_specs=[
            pl.BlockSpec((_C, _NB, _LANES), lambda i: (0, i, 0)),
            pl.BlockSpec((1, 1), lambda i: (0, 0)),
        ],
        out_shape=[
            jax.ShapeDtypeStruct((_C, _ROWS, _LANES), jnp.int32),
            jax.ShapeDtypeStruct((1, 1), jnp.float32),
        ],
    )(logits, lab)


def _sc_hist(idx_flat):
    info = plsc.get_sparse_core_info()
    nc = info.num_cores

    mesh = plsc.VectorSubcoreMesh(core_axis_name="c", subcore_axis_name="s")

    @functools.partial(
        pl.kernel,
        mesh=mesh,
        out_type=jax.ShapeDtypeStruct((_W * _T,), jnp.float32),
        scratch_types=[
            pltpu.VMEM((_T,), jnp.float32),
            pltpu.VMEM((_K,), jnp.int32),
        ],
        compiler_params=pltpu.CompilerParams(needs_layout_passes=False),
    )
    def hist_kernel(idx_hbm, out_hbm, hist_v, buf_v):
        wid = lax.axis_index("s") * nc + lax.axis_index("c")
        base = wid * _PER_W

        def zero_body(i, carry):
            hist_v[pl.ds(i * 16, 16)] = jnp.zeros((16,), jnp.float32)
            return carry

        lax.fori_loop(0, _T // 16, zero_body, 0, unroll=8)

        ones = jnp.full((16,), 1.0, jnp.float32)

        def chunk_body(ch, carry):
            pltpu.sync_copy(idx_hbm.at[pl.ds(base + ch * _K, _K)], buf_v)

            def vec_body(j, c2):
                iv = buf_v[pl.ds(j * 16, 16)]
                plsc.addupdate_scatter(hist_v, [iv], ones)
                return c2

            lax.fori_loop(0, _K // 16, vec_body, 0, unroll=8)
            return carry

        lax.fori_loop(0, _NCH, chunk_body, 0)

        pltpu.sync_copy(hist_v, out_hbm.at[pl.ds(wid * _T, _T)])

    return hist_kernel(idx_flat)


def _tc_final_body(hist_ref, nll_ref, out_ref):
    h = jnp.sum(hist_ref[...], axis=0)
    f = h[_C:]
    n = h[:_C] + f
    rows = lax.broadcasted_iota(jnp.int32, (_B, _B), 0)
    cols = lax.broadcasted_iota(jnp.int32, (_B, _B), 1)
    tri = (rows >= cols).astype(jnp.float32)
    dn = (((1,), (0,)), ((), ()))
    ncum = lax.dot_general(n, tri, dn, precision=lax.Precision.HIGHEST,
                           preferred_element_type=jnp.float32)
    fcum = lax.dot_general(f, tri, dn, precision=lax.Precision.HIGHEST,
                           preferred_element_type=jnp.float32)
    g = fcum[:, 0:1]
    denom = jnp.maximum(g + ncum - fcum, 1.0)
    jac = jnp.where(ncum > 0, 1.0 - (g - fcum) / denom, 0.0)
    loss_c = (jnp.sum(jac, axis=1, keepdims=True) - 0.5) / _B
    present = (g > 0).astype(jnp.float32)
    lov = jnp.sum(loss_c * present) / jnp.maximum(jnp.sum(present), 1.0)
    out_ref[...] = nll_ref[...] / _P + lov


def _tc_final(hist, nll):
    return pl.pallas_call(
        _tc_final_body,
        grid=(1,),
        in_specs=[
            pl.BlockSpec((_W, 2 * _C, _B), lambda i: (0, 0, 0)),
            pl.BlockSpec((1, 1), lambda i: (0, 0)),
        ],
        out_specs=pl.BlockSpec((1, 1), lambda i: (0, 0)),
        out_shape=jax.ShapeDtypeStruct((1, 1), jnp.float32),
    )(hist, nll)


def kernel(prediction, labels):
    logits = prediction.reshape(_C, _ROWS, _LANES)
    lab = labels.reshape(_ROWS, _LANES)
    idx, nll = _tc_prep(logits, lab)
    hist = _sc_hist(idx.reshape(-1))
    out = _tc_final(hist.reshape(_W, 2 * _C, _B), nll)
    return out[0, 0]


# speedup vs baseline: 71.6610x; 1.1614x over previous
import functools

import jax
import jax.numpy as jnp
from jax import lax
from jax.experimental import pallas as pl
from jax.experimental.pallas import tpu as pltpu
from jax.experimental.pallas import tpu_sc as plsc

_C = 20
_CP = _C // 2
_B = 1024
_T = 2 * _C * _B
_LANES = 128
_ROWS = 240 * 180 * 32 // _LANES
_P = _ROWS * _LANES
_NB = 120
_GRID1 = _ROWS // _NB
_W = 32
_UR = 216
_NU = _CP * _ROWS // _UR
_KMAX = (_NU + _W - 1) // _W


def _tc_prep_body(x_ref, lab_ref, idx_ref, nll_ref):
    x = x_ref[...]
    lab = lab_ref[...]
    ex = jnp.exp(x)
    s = jnp.sum(ex, axis=0)
    lse = jnp.log(s)
    rb = _B / s
    xl = jnp.zeros_like(s)
    fgs = []
    for c in range(_C):
        fg = lab == c
        fgs.append(fg)
        xl = jnp.where(fg, x[c], xl)

    @pl.when(pl.program_id(0) == 0)
    def _():
        nll_ref[...] = jnp.zeros((1, 1), jnp.float32)

    nll_ref[...] += jnp.sum(lse - xl).reshape(1, 1)

    for cp in range(_CP):
        c0, c1 = 2 * cp, 2 * cp + 1
        t0 = jnp.minimum((ex[c0] * rb).astype(jnp.int32), _B - 1)
        t1 = jnp.minimum((ex[c1] * rb).astype(jnp.int32), _B - 1)
        i0 = jnp.where(fgs[c0], (_C * _B + c0 * _B + _B - 1) - t0,
                       c0 * _B + t0)
        i1 = jnp.where(fgs[c1], (_C * _B + c1 * _B + _B - 1) - t1,
                       c1 * _B + t1)
        idx_ref[cp] = i0 | (i1 << 16)


def _tc_prep(logits, lab):
    return pl.pallas_call(
        _tc_prep_body,
        grid=(_GRID1,),
        in_specs=[
            pl.BlockSpec((_C, _NB, _LANES), lambda i: (0, i, 0)),
            pl.BlockSpec((_NB, _LANES), lambda i: (i, 0)),
        ],
        out_specs=[
            pl.BlockSpec((_CP, _NB, _LANES), lambda i: (0, i, 0)),
            pl.BlockSpec((1, 1), lambda i: (0, 0)),
        ],
        out_shape=[
            jax.ShapeDtypeStruct((_CP, _ROWS, _LANES), jnp.int32),
            jax.ShapeDtypeStruct((1, 1), jnp.float32),
        ],
    )(logits, lab)


def _sc_hist(idx_packed):
    info = plsc.get_sparse_core_info()
    nc = info.num_cores

    mesh = plsc.VectorSubcoreMesh(core_axis_name="c", subcore_axis_name="s")

    @functools.partial(
        pl.kernel,
        mesh=mesh,
        out_type=jax.ShapeDtypeStruct((_W, 2 * _C, _B), jnp.float32),
        scratch_types=[
            pltpu.VMEM((2 * _C, _B), jnp.float32),
            pltpu.VMEM((2, _UR, _LANES), jnp.int32),
            pltpu.SemaphoreType.DMA,
            pltpu.SemaphoreType.DMA,
        ],
        compiler_params=pltpu.CompilerParams(needs_layout_passes=False),
    )
    def hist_kernel(idx_hbm, out_hbm, hist_v, buf_v, sem0, sem1):
        wid = lax.axis_index("s") * nc + lax.axis_index("c")

        def zero_body(i, carry):
            hist_v[lax.div(i, jnp.int32(_B // 16)),
                   pl.ds(lax.rem(i, jnp.int32(_B // 16)) * 16, 16)] = (
                jnp.zeros((16,), jnp.float32))
            return carry

        lax.fori_loop(0, _T // 16, zero_body, 0, unroll=8)

        ones = jnp.full((16,), 1.0, jnp.float32)
        sems = [sem0, sem1]
        blocks_per_cp = _ROWS // _UR

        def unit_start(k):
            u = wid + _W * k

            @pl.when(u < _NU)
            def _():
                cp = lax.div(u, jnp.int32(blocks_per_cp))
                rb0 = lax.rem(u, jnp.int32(blocks_per_cp)) * _UR
                pltpu.make_async_copy(
                    idx_hbm.at[cp, pl.ds(rb0, _UR)],
                    buf_v.at[k % 2],
                    sems[k % 2],
                ).start()

        def unit_finish(k):
            u = wid + _W * k

            @pl.when(u < _NU)
            def _():
                cp = lax.div(u, jnp.int32(blocks_per_cp))
                rb0 = lax.rem(u, jnp.int32(blocks_per_cp)) * _UR
                pltpu.make_async_copy(
                    idx_hbm.at[cp, pl.ds(rb0, _UR)],
                    buf_v.at[k % 2],
                    sems[k % 2],
                ).wait()
                slot = buf_v.at[k % 2]

                def row_body(r, carry):
                    for q in range(_LANES // 16):
                        w = slot[r, pl.ds(q * 16, 16)]
                        lo = w & 0xFFFF
                        hi = lax.shift_right_logical(w, 16)
                        plsc.addupdate_scatter(
                            hist_v,
                            [lax.shift_right_logical(lo, 10), lo & (_B - 1)],
                            ones)
                        plsc.addupdate_scatter(
                            hist_v,
                            [lax.shift_right_logical(hi, 10), hi & (_B - 1)],
                            ones)
                    return carry

                lax.fori_loop(0, _UR, row_body, 0)

        unit_start(0)
        for k in range(_KMAX):
            if k + 1 < _KMAX:
                unit_start(k + 1)
            unit_finish(k)

        pltpu.sync_copy(hist_v, out_hbm.at[wid])

    return hist_kernel(idx_packed)


def _tc_final_body(hist_ref, nll_ref, out_ref):
    h = jnp.sum(hist_ref[...], axis=0)
    f = h[_C:]
    n = h[:_C] + f
    rows = lax.broadcasted_iota(jnp.int32, (_B, _B), 0)
    cols = lax.broadcasted_iota(jnp.int32, (_B, _B), 1)
    tri = (rows >= cols).astype(jnp.float32)
    dn = (((1,), (0,)), ((), ()))
    ncum = lax.dot_general(n, tri, dn, precision=lax.Precision.HIGHEST,
                           preferred_element_type=jnp.float32)
    fcum = lax.dot_general(f, tri, dn, precision=lax.Precision.HIGHEST,
                           preferred_element_type=jnp.float32)
    g = fcum[:, 0:1]
    denom = jnp.maximum(g + ncum - fcum, 1.0)
    jac = jnp.where(ncum > 0, 1.0 - (g - fcum) / denom, 0.0)
    loss_c = (jnp.sum(jac, axis=1, keepdims=True) - 0.5) / _B
    present = (g > 0).astype(jnp.float32)
    lov = jnp.sum(loss_c * present) / jnp.maximum(jnp.sum(present), 1.0)
    out_ref[...] = nll_ref[...] / _P + lov


def _tc_final(hist, nll):
    return pl.pallas_call(
        _tc_final_body,
        grid=(1,),
        in_specs=[
            pl.BlockSpec((_W, 2 * _C, _B), lambda i: (0, 0, 0)),
            pl.BlockSpec((1, 1), lambda i: (0, 0)),
        ],
        out_specs=pl.BlockSpec((1, 1), lambda i: (0, 0)),
        out_shape=jax.ShapeDtypeStruct((1, 1), jnp.float32),
    )(hist, nll)


def kernel(prediction, labels):
    logits = prediction.reshape(_C, _ROWS, _LANES)
    lab = labels.reshape(_ROWS, _LANES)
    idx, nll = _tc_prep(logits, lab)
    hist = _sc_hist(idx)
    out = _tc_final(hist, nll)
    return out[0, 0]


# speedup vs baseline: 79.5024x; 1.1094x over previous
import functools

import jax
import jax.numpy as jnp
from jax import lax
from jax.experimental import pallas as pl
from jax.experimental.pallas import tpu as pltpu
from jax.experimental.pallas import tpu_sc as plsc

_C = 20
_CP = _C // 2
_B = 128
_T = 2 * _C * _B
_L = 16
_LANES = 128
_ROWS = 240 * 180 * 32 // _LANES
_P = _ROWS * _LANES
_NB = 120
_GRID1 = _ROWS // _NB
_W = 32
_UR = 120
_UBLK = _ROWS // _UR
_NU = _CP * _UBLK
_KMAX = (_NU + _W - 1) // _W


def _tc_prep_body(x_ref, lab_ref, idx_ref, nll_ref):
    x = x_ref[...]
    lab = lab_ref[...]
    ex = jnp.exp(x)
    s = jnp.sum(ex, axis=0)
    lse = jnp.log(s)
    rb = _B / s
    xl = jnp.zeros_like(s)
    fgs = []
    for c in range(_C):
        fg = lab == c
        fgs.append(fg)
        xl = jnp.where(fg, x[c], xl)

    @pl.when(pl.program_id(0) == 0)
    def _():
        nll_ref[...] = jnp.zeros((1, 1), jnp.float32)

    nll_ref[...] += jnp.sum(lse - xl).reshape(1, 1)

    for cp in range(_CP):
        c0, c1 = 2 * cp, 2 * cp + 1
        t0 = jnp.minimum((ex[c0] * rb).astype(jnp.int32), _B - 1)
        t1 = jnp.minimum((ex[c1] * rb).astype(jnp.int32), _B - 1)
        i0 = jnp.where(fgs[c0], (_C * _B + c0 * _B + _B - 1) - t0,
                       c0 * _B + t0)
        i1 = jnp.where(fgs[c1], (_C * _B + c1 * _B + _B - 1) - t1,
                       c1 * _B + t1)
        idx_ref[cp] = i0 | (i1 << 16)


def _tc_prep(logits, lab):
    return pl.pallas_call(
        _tc_prep_body,
        grid=(_GRID1,),
        in_specs=[
            pl.BlockSpec((_C, _NB, _LANES), lambda i: (0, i, 0)),
            pl.BlockSpec((_NB, _LANES), lambda i: (i, 0)),
        ],
        out_specs=[
            pl.BlockSpec((_CP, _NB, _LANES), lambda i: (0, i, 0)),
            pl.BlockSpec((1, 1), lambda i: (0, 0)),
        ],
        out_shape=[
            jax.ShapeDtypeStruct((_CP, _ROWS, _LANES), jnp.int32),
            jax.ShapeDtypeStruct((1, 1), jnp.float32),
        ],
    )(logits, lab)


def _sc_hist(idx_packed):
    info = plsc.get_sparse_core_info()
    nc = info.num_cores

    mesh = plsc.VectorSubcoreMesh(core_axis_name="c", subcore_axis_name="s")

    @functools.partial(
        pl.kernel,
        mesh=mesh,
        out_type=jax.ShapeDtypeStruct((_W, 2 * _C, _B), jnp.float32),
        scratch_types=[
            pltpu.VMEM((_T * _L,), jnp.float32),
            pltpu.VMEM((2 * _C, _B), jnp.float32),
            pltpu.VMEM((2, _UR, _LANES), jnp.int32),
            pltpu.SemaphoreType.DMA,
            pltpu.SemaphoreType.DMA,
        ],
        compiler_params=pltpu.CompilerParams(needs_layout_passes=False),
    )
    def hist_kernel(idx_hbm, out_hbm, hist_v, merged_v, buf_v, sem0, sem1):
        wid = lax.axis_index("s") * nc + lax.axis_index("c")
        lane = lax.iota(jnp.int32, 16)

        def zero_body(i, carry):
            hist_v[pl.ds(i * 16, 16)] = jnp.zeros((16,), jnp.float32)
            return carry

        lax.fori_loop(0, _T * _L // 16, zero_body, 0, unroll=8)

        ones = jnp.full((16,), 1.0, jnp.float32)
        sems = [sem0, sem1]

        def unit_start(k):
            u = wid + _W * k

            @pl.when(u < _NU)
            def _():
                cp = lax.div(u, jnp.int32(_UBLK))
                rb0 = lax.rem(u, jnp.int32(_UBLK)) * _UR
                pltpu.make_async_copy(
                    idx_hbm.at[cp, pl.ds(rb0, _UR)],
                    buf_v.at[k % 2],
                    sems[k % 2],
                ).start()

        def unit_finish(k):
            u = wid + _W * k

            @pl.when(u < _NU)
            def _():
                cp = lax.div(u, jnp.int32(_UBLK))
                rb0 = lax.rem(u, jnp.int32(_UBLK)) * _UR
                pltpu.make_async_copy(
                    idx_hbm.at[cp, pl.ds(rb0, _UR)],
                    buf_v.at[k % 2],
                    sems[k % 2],
                ).wait()
                slot = buf_v.at[k % 2]

                def row_body(r, carry):
                    for q in range(_LANES // 16):
                        w = slot[r, pl.ds(q * 16, 16)]
                        lo = w & 0xFFFF
                        hi = lax.shift_right_logical(w, 16)
                        plsc.addupdate_scatter(
                            hist_v, [(lo << 4) | lane], ones)
                        plsc.addupdate_scatter(
                            hist_v, [(hi << 4) | lane], ones)
                    return carry

                lax.fori_loop(0, _UR, row_body, 0)

        unit_start(0)
        for k in range(_KMAX):
            if k + 1 < _KMAX:
                unit_start(k + 1)
            unit_finish(k)

        def merge_body(r, carry):
            for q in range(_LANES // 16):
                base = ((r * _LANES + q * 16) + lane) * 16
                acc = jnp.zeros((16,), jnp.float32)
                for l in range(_L):
                    acc = acc + plsc.load_gather(hist_v, [base + l])
                merged_v[r, pl.ds(q * 16, 16)] = acc
            return carry

        lax.fori_loop(0, 2 * _C, merge_body, 0)

        pltpu.sync_copy(merged_v, out_hbm.at[wid])

    return hist_kernel(idx_packed)


def _tc_final_body(hist_ref, nll_ref, out_ref):
    h = jnp.sum(hist_ref[...], axis=0)
    f = h[_C:]
    n = h[:_C] + f
    rows = lax.broadcasted_iota(jnp.int32, (_B, _B), 0)
    cols = lax.broadcasted_iota(jnp.int32, (_B, _B), 1)
    tri = (rows >= cols).astype(jnp.float32)
    dn = (((1,), (0,)), ((), ()))
    ncum = lax.dot_general(n, tri, dn, precision=lax.Precision.HIGHEST,
                           preferred_element_type=jnp.float32)
    fcum = lax.dot_general(f, tri, dn, precision=lax.Precision.HIGHEST,
                           preferred_element_type=jnp.float32)
    g = fcum[:, 0:1]
    denom = jnp.maximum(g + ncum - fcum, 1.0)
    jac = jnp.where(ncum > 0, 1.0 - (g - fcum) / denom, 0.0)
    loss_c = (jnp.sum(jac, axis=1, keepdims=True) - 0.5) / _B
    present = (g > 0).astype(jnp.float32)
    lov = jnp.sum(loss_c * present) / jnp.maximum(jnp.sum(present), 1.0)
    out_ref[...] = nll_ref[...] / _P + lov


def _tc_final(hist, nll):
    return pl.pallas_call(
        _tc_final_body,
        grid=(1,),
        in_specs=[
            pl.BlockSpec((_W, 2 * _C, _B), lambda i: (0, 0, 0)),
            pl.BlockSpec((1, 1), lambda i: (0, 0)),
        ],
        out_specs=pl.BlockSpec((1, 1), lambda i: (0, 0)),
        out_shape=jax.ShapeDtypeStruct((1, 1), jnp.float32),
    )(hist, nll)


def kernel(prediction, labels):
    logits = prediction.reshape(_C, _ROWS, _LANES)
    lab = labels.reshape(_ROWS, _LANES)
    idx, nll = _tc_prep(logits, lab)
    hist = _sc_hist(idx)
    out = _tc_final(hist, nll)
    return out[0, 0]


# speedup vs baseline: 151.7790x; 1.9091x over previous
import functools

import jax
import jax.numpy as jnp
from jax import lax
from jax.experimental import pallas as pl
from jax.experimental.pallas import tpu as pltpu
from jax.experimental.pallas import tpu_sc as plsc

_C = 20
_CP = _C // 2
_B = 128
_T = 2 * _C * _B
_L = 16
_LANES = 128
_ROWS = 240 * 180 * 32 // _LANES
_P = _ROWS * _LANES
_MINOR = 240
_VROWS = 180 * 32
_NBS = 64
_GRID1 = _VROWS // _NBS
_OMINOR = 256
_DUMMY = _T
_DUMMYW = _DUMMY | (_DUMMY << 16)
_W = 32
_UR = 72
_UBLK = _VROWS // _UR
_NU = _CP * _UBLK
_KMAX = _NU // _W


def _tc_prep_body(x_ref, lab_ref, idx_ref, nll_ref):
    x = x_ref[...]
    lab = lab_ref[...]
    ex = jnp.exp(x)
    s = jnp.sum(ex, axis=0)
    lse = jnp.log(s)
    rb = _B / s
    xl = jnp.zeros_like(s)
    fgs = []
    for c in range(_C):
        fg = lab == c
        fgs.append(fg)
        xl = jnp.where(fg, x[c], xl)

    @pl.when(pl.program_id(0) == 0)
    def _():
        nll_ref[...] = jnp.zeros((1, 1), jnp.float32)

    nll_ref[...] += jnp.sum(lse - xl).reshape(1, 1)

    for cp in range(_CP):
        c0, c1 = 2 * cp, 2 * cp + 1
        t0 = jnp.minimum((ex[c0] * rb).astype(jnp.int32), _B - 1)
        t1 = jnp.minimum((ex[c1] * rb).astype(jnp.int32), _B - 1)
        i0 = jnp.where(fgs[c0], (_C * _B + c0 * _B + _B - 1) - t0,
                       c0 * _B + t0)
        i1 = jnp.where(fgs[c1], (_C * _B + c1 * _B + _B - 1) - t1,
                       c1 * _B + t1)
        iw = i0 | (i1 << 16)
        pad = jnp.full((_NBS, _OMINOR - _MINOR), _DUMMYW, jnp.int32)
        idx_ref[cp] = jnp.concatenate([iw, pad], axis=1)


def _tc_prep(logits, lab):
    return pl.pallas_call(
        _tc_prep_body,
        grid=(_GRID1,),
        in_specs=[
            pl.BlockSpec((_C, _NBS, _MINOR), lambda i: (0, i, 0)),
            pl.BlockSpec((_NBS, _MINOR), lambda i: (i, 0)),
        ],
        out_specs=[
            pl.BlockSpec((_CP, _NBS, _OMINOR), lambda i: (0, i, 0)),
            pl.BlockSpec((1, 1), lambda i: (0, 0)),
        ],
        out_shape=[
            jax.ShapeDtypeStruct((_CP, _VROWS, _OMINOR), jnp.int32),
            jax.ShapeDtypeStruct((1, 1), jnp.float32),
        ],
    )(logits, lab)


def _sc_hist(idx_packed):
    info = plsc.get_sparse_core_info()
    nc = info.num_cores

    mesh = plsc.VectorSubcoreMesh(core_axis_name="c", subcore_axis_name="s")

    @functools.partial(
        pl.kernel,
        mesh=mesh,
        out_type=jax.ShapeDtypeStruct((_W, 2 * _C, _B), jnp.float32),
        scratch_types=[
            pltpu.VMEM((_T * _L + _L,), jnp.float32),
            pltpu.VMEM((2 * _C, _B), jnp.float32),
            pltpu.VMEM((2, _UR, _OMINOR), jnp.int32),
            pltpu.SemaphoreType.DMA,
            pltpu.SemaphoreType.DMA,
        ],
        compiler_params=pltpu.CompilerParams(needs_layout_passes=False),
    )
    def hist_kernel(idx_hbm, out_hbm, hist_v, merged_v, buf_v, sem0, sem1):
        wid = lax.axis_index("s") * nc + lax.axis_index("c")
        lane = lax.iota(jnp.int32, 16)

        def zero_body(i, carry):
            hist_v[pl.ds(i * 16, 16)] = jnp.zeros((16,), jnp.float32)
            return carry

        lax.fori_loop(0, (_T * _L + _L) // 16, zero_body, 0, unroll=8)

        ones = jnp.full((16,), 1.0, jnp.float32)
        sems = [sem0, sem1]

        def unit_start(k):
            u = wid + _W * k

            @pl.when(u < _NU)
            def _():
                cp = lax.div(u, jnp.int32(_UBLK))
                rb0 = lax.rem(u, jnp.int32(_UBLK)) * _UR
                pltpu.make_async_copy(
                    idx_hbm.at[cp, pl.ds(rb0, _UR)],
                    buf_v.at[k % 2],
                    sems[k % 2],
                ).start()

        def unit_finish(k):
            u = wid + _W * k

            @pl.when(u < _NU)
            def _():
                cp = lax.div(u, jnp.int32(_UBLK))
                rb0 = lax.rem(u, jnp.int32(_UBLK)) * _UR
                pltpu.make_async_copy(
                    idx_hbm.at[cp, pl.ds(rb0, _UR)],
                    buf_v.at[k % 2],
                    sems[k % 2],
                ).wait()
                slot = buf_v.at[k % 2]

                def row_body(r, carry):
                    for q in range(_OMINOR // 16):
                        w = slot[r, pl.ds(q * 16, 16)]
                        lo = w & 0xFFFF
                        hi = lax.shift_right_logical(w, 16)
                        plsc.addupdate_scatter(
                            hist_v, [(lo << 4) | lane], ones)
                        plsc.addupdate_scatter(
                            hist_v, [(hi << 4) | lane], ones)
                    return carry

                lax.fori_loop(0, _UR, row_body, 0)

        unit_start(0)
        for k in range(_KMAX):
            if k + 1 < _KMAX:
                unit_start(k + 1)
            unit_finish(k)

        def merge_body(r, carry):
            for q in range(_LANES // 16):
                base = ((r * _LANES + q * 16) + lane) * 16
                acc = jnp.zeros((16,), jnp.float32)
                for l in range(_L):
                    acc = acc + plsc.load_gather(hist_v, [base + l])
                merged_v[r, pl.ds(q * 16, 16)] = acc
            return carry

        lax.fori_loop(0, 2 * _C, merge_body, 0)

        pltpu.sync_copy(merged_v, out_hbm.at[wid])

    return hist_kernel(idx_packed)


def _tc_final_body(hist_ref, nll_ref, out_ref):
    h = jnp.sum(hist_ref[...], axis=0)
    f = h[_C:]
    n = h[:_C] + f
    rows = lax.broadcasted_iota(jnp.int32, (_B, _B), 0)
    cols = lax.broadcasted_iota(jnp.int32, (_B, _B), 1)
    tri = (rows >= cols).astype(jnp.float32)
    dn = (((1,), (0,)), ((), ()))
    ncum = lax.dot_general(n, tri, dn, precision=lax.Precision.HIGHEST,
                           preferred_element_type=jnp.float32)
    fcum = lax.dot_general(f, tri, dn, precision=lax.Precision.HIGHEST,
                           preferred_element_type=jnp.float32)
    g = fcum[:, 0:1]
    denom = jnp.maximum(g + ncum - fcum, 1.0)
    jac = jnp.where(ncum > 0, 1.0 - (g - fcum) / denom, 0.0)
    loss_c = (jnp.sum(jac, axis=1, keepdims=True) - 0.5) / _B
    present = (g > 0).astype(jnp.float32)
    lov = jnp.sum(loss_c * present) / jnp.maximum(jnp.sum(present), 1.0)
    out_ref[...] = nll_ref[...] / _P + lov


def _tc_final(hist, nll):
    return pl.pallas_call(
        _tc_final_body,
        grid=(1,),
        in_specs=[
            pl.BlockSpec((_W, 2 * _C, _B), lambda i: (0, 0, 0)),
            pl.BlockSpec((1, 1), lambda i: (0, 0)),
        ],
        out_specs=pl.BlockSpec((1, 1), lambda i: (0, 0)),
        out_shape=jax.ShapeDtypeStruct((1, 1), jnp.float32),
    )(hist, nll)


def kernel(prediction, labels):
    logits = jnp.transpose(prediction, (0, 1, 3, 4, 2)).reshape(
        _C, 180 * 32, _MINOR)
    lab = jnp.transpose(labels, (0, 2, 3, 1)).reshape(180 * 32, _MINOR)
    idx, nll = _tc_prep(logits, lab)
    hist = _sc_hist(idx)
    out = _tc_final(hist, nll)
    return out[0, 0]


# speedup vs baseline: 283.7588x; 1.8696x over previous
import functools

import jax
import jax.numpy as jnp
from jax import lax
from jax.experimental import pallas as pl
from jax.experimental.pallas import tpu as pltpu
from jax.experimental.pallas import tpu_sc as plsc

_C = 20
_CP = _C // 2
_B = 128
_T = 2 * _C * _B
_L = 16
_LANES = 128
_ROWS = 240 * 180 * 32 // _LANES
_P = _ROWS * _LANES
_MINOR = 240
_VROWS = 180 * 32
_NBS = 64
_GRID1 = _VROWS // _NBS
_OMINOR = 256
_DUMMY = _T
_DUMMYW = _DUMMY | (_DUMMY << 16)
_W = 32
_UR = 72
_UBLK = _VROWS // _UR
_NU = _CP * _UBLK
_KMAX = _NU // _W


def _tc_prep_body(x_ref, lab_ref, idx_ref, nll_ref):
    x = x_ref[...]
    lab = lab_ref[...]
    ex = jnp.exp(x)
    s = jnp.sum(ex, axis=0)
    lse = jnp.log(s)
    rb = _B / s
    xl = jnp.zeros_like(s)
    fgs = []
    for c in range(_C):
        fg = lab == c
        fgs.append(fg)
        xl = jnp.where(fg, x[c], xl)

    @pl.when(pl.program_id(0) == 0)
    def _():
        nll_ref[...] = jnp.zeros((1, 1), jnp.float32)

    nll_ref[...] += jnp.sum(lse - xl).reshape(1, 1)

    for cp in range(_CP):
        c0, c1 = 2 * cp, 2 * cp + 1
        t0 = jnp.minimum((ex[c0] * rb).astype(jnp.int32), _B - 1)
        t1 = jnp.minimum((ex[c1] * rb).astype(jnp.int32), _B - 1)
        i0 = jnp.where(fgs[c0], (_C * _B + c0 * _B + _B - 1) - t0,
                       c0 * _B + t0)
        i1 = jnp.where(fgs[c1], (_C * _B + c1 * _B + _B - 1) - t1,
                       c1 * _B + t1)
        iw = i0 | (i1 << 16)
        pad = jnp.full((_NBS, _OMINOR - _MINOR), _DUMMYW, jnp.int32)
        idx_ref[cp] = jnp.concatenate([iw, pad], axis=1)


def _tc_prep(logits, lab):
    return pl.pallas_call(
        _tc_prep_body,
        grid=(_GRID1,),
        in_specs=[
            pl.BlockSpec((_C, _NBS, _MINOR), lambda i: (0, i, 0)),
            pl.BlockSpec((_NBS, _MINOR), lambda i: (i, 0)),
        ],
        out_specs=[
            pl.BlockSpec((_CP, _NBS, _OMINOR), lambda i: (0, i, 0)),
            pl.BlockSpec((1, 1), lambda i: (0, 0)),
        ],
        out_shape=[
            jax.ShapeDtypeStruct((_CP, _VROWS, _OMINOR), jnp.int32),
            jax.ShapeDtypeStruct((1, 1), jnp.float32),
        ],
    )(logits, lab)


def _sc_hist(idx_packed):
    info = plsc.get_sparse_core_info()
    nc = info.num_cores

    mesh = plsc.VectorSubcoreMesh(core_axis_name="c", subcore_axis_name="s")

    @functools.partial(
        pl.kernel,
        mesh=mesh,
        out_type=jax.ShapeDtypeStruct((_W, 2 * _C, _B), jnp.float32),
        scratch_types=[
            pltpu.VMEM((_T * _L + _L,), jnp.float32),
            pltpu.VMEM((2 * _C, _B), jnp.float32),
            pltpu.VMEM((2, _UR, _OMINOR), jnp.int32),
            pltpu.SemaphoreType.DMA,
            pltpu.SemaphoreType.DMA,
        ],
        compiler_params=pltpu.CompilerParams(needs_layout_passes=False),
    )
    def hist_kernel(idx_hbm, out_hbm, hist_v, merged_v, buf_v, sem0, sem1):
        wid = lax.axis_index("s") * nc + lax.axis_index("c")
        lane = lax.iota(jnp.int32, 16)

        @plsc.parallel_loop(0, (_T * _L + _L) // 16, unroll=4)
        def zero_body(i):
            hist_v[pl.ds(i * 16, 16)] = jnp.zeros((16,), jnp.float32)

        ones = jnp.full((16,), 1.0, jnp.float32)
        sems = [sem0, sem1]

        def unit_start(k, slot_i):
            u = wid + _W * k
            cp = lax.div(u, jnp.int32(_UBLK))
            rb0 = lax.rem(u, jnp.int32(_UBLK)) * _UR
            pltpu.make_async_copy(
                idx_hbm.at[cp, pl.ds(rb0, _UR)],
                buf_v.at[slot_i],
                sems[slot_i],
            ).start()

        def unit_process(slot_i):
            pltpu.make_async_copy(
                idx_hbm.at[0, pl.ds(0, _UR)],
                buf_v.at[slot_i],
                sems[slot_i],
            ).wait()
            slot = buf_v.at[slot_i]

            @plsc.parallel_loop(0, _UR)
            def row_body(r):
                for q in range(_OMINOR // 16):
                    w = slot[r, pl.ds(q * 16, 16)]
                    lo = w & 0xFFFF
                    hi = lax.shift_right_logical(w, 16)
                    plsc.addupdate_scatter(
                        hist_v, [(lo << 4) | lane], ones)
                    plsc.addupdate_scatter(
                        hist_v, [(hi << 4) | lane], ones)

        unit_start(0, 0)

        def pair_body(j, carry):
            k = 2 * j
            unit_start(k + 1, 1)
            unit_process(0)

            @pl.when(k + 2 < _KMAX)
            def _():
                unit_start(k + 2, 0)

            unit_process(1)
            return carry

        lax.fori_loop(0, _KMAX // 2, pair_body, 0)
        unit_process(0)

        @plsc.parallel_loop(0, 2 * _C)
        def merge_body(r):
            for q in range(_LANES // 16):
                base = ((r * _LANES + q * 16) + lane) * 16
                acc = jnp.zeros((16,), jnp.float32)
                for l in range(_L):
                    acc = acc + plsc.load_gather(hist_v, [base + l])
                merged_v[r, pl.ds(q * 16, 16)] = acc

        pltpu.sync_copy(merged_v, out_hbm.at[wid])

    return hist_kernel(idx_packed)


def _tc_final_body(hist_ref, nll_ref, out_ref):
    h = jnp.sum(hist_ref[...], axis=0)
    f = h[_C:]
    n = h[:_C] + f
    rows = lax.broadcasted_iota(jnp.int32, (_B, _B), 0)
    cols = lax.broadcasted_iota(jnp.int32, (_B, _B), 1)
    tri = (rows >= cols).astype(jnp.float32)
    dn = (((1,), (0,)), ((), ()))
    ncum = lax.dot_general(n, tri, dn, precision=lax.Precision.HIGHEST,
                           preferred_element_type=jnp.float32)
    fcum = lax.dot_general(f, tri, dn, precision=lax.Precision.HIGHEST,
                           preferred_element_type=jnp.float32)
    g = fcum[:, 0:1]
    denom = jnp.maximum(g + ncum - fcum, 1.0)
    jac = jnp.where(ncum > 0, 1.0 - (g - fcum) / denom, 0.0)
    loss_c = (jnp.sum(jac, axis=1, keepdims=True) - 0.5) / _B
    present = (g > 0).astype(jnp.float32)
    lov = jnp.sum(loss_c * present) / jnp.maximum(jnp.sum(present), 1.0)
    out_ref[...] = nll_ref[...] / _P + lov


def _tc_final(hist, nll):
    return pl.pallas_call(
        _tc_final_body,
        grid=(1,),
        in_specs=[
            pl.BlockSpec((_W, 2 * _C, _B), lambda i: (0, 0, 0)),
            pl.BlockSpec((1, 1), lambda i: (0, 0)),
        ],
        out_specs=pl.BlockSpec((1, 1), lambda i: (0, 0)),
        out_shape=jax.ShapeDtypeStruct((1, 1), jnp.float32),
    )(hist, nll)


def kernel(prediction, labels):
    logits = jnp.transpose(prediction, (0, 1, 3, 4, 2)).reshape(
        _C, 180 * 32, _MINOR)
    lab = jnp.transpose(labels, (0, 2, 3, 1)).reshape(180 * 32, _MINOR)
    idx, nll = _tc_prep(logits, lab)
    hist = _sc_hist(idx)
    out = _tc_final(hist, nll)
    return out[0, 0]


# speedup vs baseline: 317.4311x; 1.1187x over previous
import functools

import jax
import jax.numpy as jnp
from jax import lax
from jax.experimental import pallas as pl
from jax.experimental.pallas import tpu as pltpu
from jax.experimental.pallas import tpu_sc as plsc

_C = 20
_CP = _C // 2
_B = 128
_T = 2 * _C * _B
_L = 16
_LANES = 128
_ROWS = 240 * 180 * 32 // _LANES
_P = _ROWS * _LANES
_MINOR = 240
_VROWS = 180 * 32
_NBS = 128
_GRID1 = _VROWS // _NBS
_OMINOR = 256
_DUMMY = _T
_DUMMYW = _DUMMY | (_DUMMY << 16)
_W = 32
_UR = 72
_UBLK = _VROWS // _UR
_NU = _CP * _UBLK
_KMAX = _NU // _W


def _tc_prep_body(x_ref, lab_ref, idx_ref, nll_ref):
    x = x_ref[...]
    lab = lab_ref[...]
    ex = jnp.exp(x)
    s = jnp.sum(ex, axis=0)
    lse = jnp.log(s)
    rb = _B / s
    xl = jnp.zeros_like(s)
    fgs = []
    for c in range(_C):
        fg = lab == c
        fgs.append(fg)
        xl = jnp.where(fg, x[c], xl)

    @pl.when(pl.program_id(0) == 0)
    def _():
        nll_ref[...] = jnp.zeros((1, 1), jnp.float32)

    nll_ref[...] += jnp.sum(lse - xl).reshape(1, 1)

    for cp in range(_CP):
        c0, c1 = 2 * cp, 2 * cp + 1
        t0 = jnp.minimum((ex[c0] * rb).astype(jnp.int32), _B - 1)
        t1 = jnp.minimum((ex[c1] * rb).astype(jnp.int32), _B - 1)
        i0 = jnp.where(fgs[c0], (_C * _B + c0 * _B + _B - 1) - t0,
                       c0 * _B + t0)
        i1 = jnp.where(fgs[c1], (_C * _B + c1 * _B + _B - 1) - t1,
                       c1 * _B + t1)
        iw = i0 | (i1 << 16)
        pad = jnp.full((_NBS, _OMINOR - _MINOR), _DUMMYW, jnp.int32)
        idx_ref[cp] = jnp.concatenate([iw, pad], axis=1)


def _tc_prep(logits, lab):
    return pl.pallas_call(
        _tc_prep_body,
        grid=(_GRID1,),
        in_specs=[
            pl.BlockSpec((_C, _NBS, _MINOR), lambda i: (0, i, 0)),
            pl.BlockSpec((_NBS, _MINOR), lambda i: (i, 0)),
        ],
        out_specs=[
            pl.BlockSpec((_CP, _NBS, _OMINOR), lambda i: (0, i, 0)),
            pl.BlockSpec((1, 1), lambda i: (0, 0)),
        ],
        out_shape=[
            jax.ShapeDtypeStruct((_CP, _VROWS, _OMINOR), jnp.int32),
            jax.ShapeDtypeStruct((1, 1), jnp.float32),
        ],
    )(logits, lab)


def _sc_hist(idx_packed):
    info = plsc.get_sparse_core_info()
    nc = info.num_cores

    mesh = plsc.VectorSubcoreMesh(core_axis_name="c", subcore_axis_name="s")

    @functools.partial(
        pl.kernel,
        mesh=mesh,
        out_type=jax.ShapeDtypeStruct((_W, 2 * _C, _B), jnp.float32),
        scratch_types=[
            pltpu.VMEM((_T * _L + _L,), jnp.float32),
            pltpu.VMEM((2 * _C, _B), jnp.float32),
            pltpu.VMEM((2, _UR, _OMINOR), jnp.int32),
            pltpu.SemaphoreType.DMA,
            pltpu.SemaphoreType.DMA,
        ],
        compiler_params=pltpu.CompilerParams(needs_layout_passes=False),
    )
    def hist_kernel(idx_hbm, out_hbm, hist_v, merged_v, buf_v, sem0, sem1):
        wid = lax.axis_index("s") * nc + lax.axis_index("c")
        lane = lax.iota(jnp.int32, 16)

        @plsc.parallel_loop(0, (_T * _L + _L) // 16, unroll=4)
        def zero_body(i):
            hist_v[pl.ds(i * 16, 16)] = jnp.zeros((16,), jnp.float32)

        ones = jnp.full((16,), 1.0, jnp.float32)
        sems = [sem0, sem1]

        def unit_start(k, slot_i):
            u = wid + _W * k
            cp = lax.div(u, jnp.int32(_UBLK))
            rb0 = lax.rem(u, jnp.int32(_UBLK)) * _UR
            pltpu.make_async_copy(
                idx_hbm.at[cp, pl.ds(rb0, _UR)],
                buf_v.at[slot_i],
                sems[slot_i],
            ).start()

        def unit_process(slot_i):
            pltpu.make_async_copy(
                idx_hbm.at[0, pl.ds(0, _UR)],
                buf_v.at[slot_i],
                sems[slot_i],
            ).wait()
            slot = buf_v.at[slot_i]

            @plsc.parallel_loop(0, _UR)
            def row_body(r):
                for q in range(_OMINOR // 16):
                    w = slot[r, pl.ds(q * 16, 16)]
                    lo = w & 0xFFFF
                    hi = lax.shift_right_logical(w, 16)
                    plsc.addupdate_scatter(
                        hist_v, [(lo << 4) | lane], ones)
                    plsc.addupdate_scatter(
                        hist_v, [(hi << 4) | lane], ones)

        unit_start(0, 0)

        def pair_body(j, carry):
            k = 2 * j
            unit_start(k + 1, 1)
            unit_process(0)

            @pl.when(k + 2 < _KMAX)
            def _():
                unit_start(k + 2, 0)

            unit_process(1)
            return carry

        lax.fori_loop(0, _KMAX // 2, pair_body, 0)
        unit_process(0)

        @plsc.parallel_loop(0, 2 * _C)
        def merge_body(r):
            for q in range(_LANES // 16):
                base = ((r * _LANES + q * 16) + lane) * 16
                acc = jnp.zeros((16,), jnp.float32)
                for l in range(_L):
                    acc = acc + plsc.load_gather(hist_v, [base + l])
                merged_v[r, pl.ds(q * 16, 16)] = acc

        pltpu.sync_copy(merged_v, out_hbm.at[wid])

    return hist_kernel(idx_packed)


def _tc_final_body(hist_ref, nll_ref, out_ref):
    h = jnp.sum(hist_ref[...], axis=0)
    f = h[_C:]
    n = h[:_C] + f
    rows = lax.broadcasted_iota(jnp.int32, (_B, _B), 0)
    cols = lax.broadcasted_iota(jnp.int32, (_B, _B), 1)
    tri = (rows >= cols).astype(jnp.float32)
    dn = (((1,), (0,)), ((), ()))
    ncum = lax.dot_general(n, tri, dn, precision=lax.Precision.HIGHEST,
                           preferred_element_type=jnp.float32)
    fcum = lax.dot_general(f, tri, dn, precision=lax.Precision.HIGHEST,
                           preferred_element_type=jnp.float32)
    g = fcum[:, 0:1]
    denom = jnp.maximum(g + ncum - fcum, 1.0)
    jac = jnp.where(ncum > 0, 1.0 - (g - fcum) / denom, 0.0)
    loss_c = (jnp.sum(jac, axis=1, keepdims=True) - 0.5) / _B
    present = (g > 0).astype(jnp.float32)
    lov = jnp.sum(loss_c * present) / jnp.maximum(jnp.sum(present), 1.0)
    out_ref[...] = nll_ref[...] / _P + lov


def _tc_final(hist, nll):
    return pl.pallas_call(
        _tc_final_body,
        grid=(1,),
        in_specs=[
            pl.BlockSpec((_W, 2 * _C, _B), lambda i: (0, 0, 0)),
            pl.BlockSpec((1, 1), lambda i: (0, 0)),
        ],
        out_specs=pl.BlockSpec((1, 1), lambda i: (0, 0)),
        out_shape=jax.ShapeDtypeStruct((1, 1), jnp.float32),
    )(hist, nll)


def kernel(prediction, labels):
    logits = jnp.transpose(prediction, (0, 1, 3, 4, 2)).reshape(
        _C, 180 * 32, _MINOR)
    lab = jnp.transpose(labels, (0, 2, 3, 1)).reshape(180 * 32, _MINOR)
    idx, nll = _tc_prep(logits, lab)
    hist = _sc_hist(idx)
    out = _tc_final(hist, nll)
    return out[0, 0]


# speedup vs baseline: 339.5640x; 1.0697x over previous
import functools

import jax
import jax.numpy as jnp
from jax import lax
from jax.experimental import pallas as pl
from jax.experimental.pallas import tpu as pltpu
from jax.experimental.pallas import tpu_sc as plsc

_C = 20
_CP = _C // 2
_B = 128
_T = 2 * _C * _B
_L = 16
_LANES = 128
_ROWS = 240 * 180 * 32 // _LANES
_P = _ROWS * _LANES
_MINOR = 240
_VROWS = 180 * 32
_S = 2
_SROWS = _VROWS // _S
_NBS = 96
_GRID1 = _SROWS // _NBS
_OMINOR = 256
_DUMMY = _T
_DUMMYW = _DUMMY | (_DUMMY << 16)
_W = 32
_UR = 72
_UBLK = _SROWS // _UR
_NU = _CP * _UBLK
_KMAX = (_NU + _W - 1) // _W


def _tc_prep_body(x_ref, lab_ref, idx_ref, nll_ref):
    x = x_ref[...]
    lab = lab_ref[...]
    ex = jnp.exp(x)
    s = jnp.sum(ex, axis=0)
    lse = jnp.log(s)
    rb = _B / s
    xl = jnp.zeros_like(s)
    fgs = []
    for c in range(_C):
        fg = lab == c
        fgs.append(fg)
        xl = jnp.where(fg, x[c], xl)

    @pl.when(pl.program_id(0) == 0)
    def _():
        nll_ref[...] = jnp.zeros((1, 1), jnp.float32)

    nll_ref[...] += jnp.sum(lse - xl).reshape(1, 1)

    for cp in range(_CP):
        c0, c1 = 2 * cp, 2 * cp + 1
        t0 = jnp.minimum((ex[c0] * rb).astype(jnp.int32), _B - 1)
        t1 = jnp.minimum((ex[c1] * rb).astype(jnp.int32), _B - 1)
        i0 = jnp.where(fgs[c0], (_C * _B + c0 * _B + _B - 1) - t0,
                       c0 * _B + t0)
        i1 = jnp.where(fgs[c1], (_C * _B + c1 * _B + _B - 1) - t1,
                       c1 * _B + t1)
        iw = i0 | (i1 << 16)
        pad = jnp.full((_NBS, _OMINOR - _MINOR), _DUMMYW, jnp.int32)
        idx_ref[cp] = jnp.concatenate([iw, pad], axis=1)


def _tc_prep(logits, lab, off):
    return pl.pallas_call(
        _tc_prep_body,
        grid=(_GRID1,),
        in_specs=[
            pl.BlockSpec((_C, _NBS, _MINOR), lambda i: (0, i + off, 0)),
            pl.BlockSpec((_NBS, _MINOR), lambda i: (i + off, 0)),
        ],
        out_specs=[
            pl.BlockSpec((_CP, _NBS, _OMINOR), lambda i: (0, i, 0)),
            pl.BlockSpec((1, 1), lambda i: (0, 0)),
        ],
        out_shape=[
            jax.ShapeDtypeStruct((_CP, _SROWS, _OMINOR), jnp.int32),
            jax.ShapeDtypeStruct((1, 1), jnp.float32),
        ],
    )(logits, lab)


def _sc_hist(idx_packed):
    info = plsc.get_sparse_core_info()
    nc = info.num_cores

    mesh = plsc.VectorSubcoreMesh(core_axis_name="c", subcore_axis_name="s")

    @functools.partial(
        pl.kernel,
        mesh=mesh,
        out_type=jax.ShapeDtypeStruct((_W, 2 * _C, _B), jnp.float32),
        scratch_types=[
            pltpu.VMEM((_T * _L + _L,), jnp.float32),
            pltpu.VMEM((2 * _C, _B), jnp.float32),
            pltpu.VMEM((2, _UR, _OMINOR), jnp.int32),
            pltpu.SemaphoreType.DMA,
            pltpu.SemaphoreType.DMA,
        ],
        compiler_params=pltpu.CompilerParams(needs_layout_passes=False),
    )
    def hist_kernel(idx_hbm, out_hbm, hist_v, merged_v, buf_v, sem0, sem1):
        wid = lax.axis_index("s") * nc + lax.axis_index("c")
        lane = lax.iota(jnp.int32, 16)

        @plsc.parallel_loop(0, (_T * _L + _L) // 16, unroll=4)
        def zero_body(i):
            hist_v[pl.ds(i * 16, 16)] = jnp.zeros((16,), jnp.float32)

        ones = jnp.full((16,), 1.0, jnp.float32)
        sems = [sem0, sem1]

        def unit_start(k, slot_i):
            u = wid + _W * k

            @pl.when(u < _NU)
            def _():
                cp = lax.div(u, jnp.int32(_UBLK))
                rb0 = lax.rem(u, jnp.int32(_UBLK)) * _UR
                pltpu.make_async_copy(
                    idx_hbm.at[cp, pl.ds(rb0, _UR)],
                    buf_v.at[slot_i],
                    sems[slot_i],
                ).start()

        def unit_process(slot_i):
            pltpu.make_async_copy(
                idx_hbm.at[0, pl.ds(0, _UR)],
                buf_v.at[slot_i],
                sems[slot_i],
            ).wait()
            slot = buf_v.at[slot_i]

            @plsc.parallel_loop(0, _UR)
            def row_body(r):
                for q in range(_OMINOR // 16):
                    w = slot[r, pl.ds(q * 16, 16)]
                    lo = w & 0xFFFF
                    hi = lax.shift_right_logical(w, 16)
                    plsc.addupdate_scatter(
                        hist_v, [(lo << 4) | lane], ones)
                    plsc.addupdate_scatter(
                        hist_v, [(hi << 4) | lane], ones)

        unit_start(0, 0)

        def pair_body(j, carry):
            k = 2 * j
            unit_start(k + 1, 1)
            unit_process(0)

            @pl.when(k + 2 < _KMAX)
            def _():
                unit_start(k + 2, 0)

            unit_process(1)
            return carry

        lax.fori_loop(0, _KMAX // 2, pair_body, 0)

        @pl.when(wid + _W * (_KMAX - 1) < _NU)
        def _():
            unit_process(0)

        @plsc.parallel_loop(0, 2 * _C)
        def merge_body(r):
            for q in range(_LANES // 16):
                base = ((r * _LANES + q * 16) + lane) * 16
                acc = jnp.zeros((16,), jnp.float32)
                for l in range(_L):
                    acc = acc + plsc.load_gather(hist_v, [base + l])
                merged_v[r, pl.ds(q * 16, 16)] = acc

        pltpu.sync_copy(merged_v, out_hbm.at[wid])

    return hist_kernel(idx_packed)


def _tc_final_body(hist_ref, hist2_ref, nll_ref, nll2_ref, out_ref):
    h = jnp.sum(hist_ref[...], axis=0) + jnp.sum(hist2_ref[...], axis=0)
    f = h[_C:]
    n = h[:_C] + f
    rows = lax.broadcasted_iota(jnp.int32, (_B, _B), 0)
    cols = lax.broadcasted_iota(jnp.int32, (_B, _B), 1)
    tri = (rows >= cols).astype(jnp.float32)
    dn = (((1,), (0,)), ((), ()))
    ncum = lax.dot_general(n, tri, dn, precision=lax.Precision.HIGHEST,
                           preferred_element_type=jnp.float32)
    fcum = lax.dot_general(f, tri, dn, precision=lax.Precision.HIGHEST,
                           preferred_element_type=jnp.float32)
    g = fcum[:, 0:1]
    denom = jnp.maximum(g + ncum - fcum, 1.0)
    jac = jnp.where(ncum > 0, 1.0 - (g - fcum) / denom, 0.0)
    loss_c = (jnp.sum(jac, axis=1, keepdims=True) - 0.5) / _B
    present = (g > 0).astype(jnp.float32)
    lov = jnp.sum(loss_c * present) / jnp.maximum(jnp.sum(present), 1.0)
    out_ref[...] = (nll_ref[...] + nll2_ref[...]) / _P + lov


def _tc_final(hist, hist2, nll, nll2):
    return pl.pallas_call(
        _tc_final_body,
        grid=(1,),
        in_specs=[
            pl.BlockSpec((_W, 2 * _C, _B), lambda i: (0, 0, 0)),
            pl.BlockSpec((_W, 2 * _C, _B), lambda i: (0, 0, 0)),
            pl.BlockSpec((1, 1), lambda i: (0, 0)),
            pl.BlockSpec((1, 1), lambda i: (0, 0)),
        ],
        out_specs=pl.BlockSpec((1, 1), lambda i: (0, 0)),
        out_shape=jax.ShapeDtypeStruct((1, 1), jnp.float32),
    )(hist, hist2, nll, nll2)


def kernel(prediction, labels):
    logits = jnp.transpose(prediction, (0, 1, 3, 4, 2)).reshape(
        _C, 180 * 32, _MINOR)
    lab = jnp.transpose(labels, (0, 2, 3, 1)).reshape(180 * 32, _MINOR)
    idx0, nll0 = _tc_prep(logits, lab, 0)
    hist0 = _sc_hist(idx0)
    idx1, nll1 = _tc_prep(logits, lab, _GRID1)
    hist1 = _sc_hist(idx1)
    out = _tc_final(hist0, hist1, nll0, nll1)
    return out[0, 0]
